# baseline (speedup 1.0000x reference)
import functools

import jax
import jax.numpy as jnp
from jax import lax
from jax.experimental import pallas as pl
from jax.experimental.pallas import tpu as pltpu
from jax.experimental.pallas import tpu_sc as plsc

BATCH = 4096
SEQ = 200
EMB_DIM = 64

NC = 2
NS = 16
NW = NC * NS

BB = BATCH // NW
NG = 8
NT = 4
NSTEPS = SEQ
NTILES = EMB_DIM // 8
TB_PAD = BB + 8


def _make_gather():
    mesh = plsc.VectorSubcoreMesh(core_axis_name="c", subcore_axis_name="s")

    @functools.partial(
        pl.kernel,
        mesh=mesh,
        out_type=jax.ShapeDtypeStruct((SEQ, NTILES, NW, 8, BB), jnp.float32),
        scratch_types=(
            [pltpu.VMEM((SEQ // 8 + 1, 8, BB), jnp.int32)]
            + [pltpu.VMEM((BB, EMB_DIM), jnp.float32)] * NG
            + [pltpu.VMEM((EMB_DIM, TB_PAD), jnp.float32)] * NT
            + [pltpu.SemaphoreType.DMA((NG,)), pltpu.SemaphoreType.DMA((NT,))]
        ),
        compiler_params=pltpu.CompilerParams(
            use_tc_tiling_on_sc=False, needs_layout_passes=False),
    )
    def gather_kernel(table_hbm, ids_hbm, out_hbm, idx_v, *bufs):
        gbufs = list(bufs[:NG])
        tbufs = list(bufs[NG:NG + NT])
        gsem, osem = bufs[NG + NT], bufs[NG + NT + 1]
        wid = lax.axis_index("s") * NC + lax.axis_index("c")
        pltpu.sync_copy(ids_hbm.at[:, wid], idx_v.at[pl.ds(0, SEQ // 8)])
        zeros16 = jnp.zeros((16,), jnp.int32)
        for s8 in range(8):
            for j in range(BB // 16):
                idx_v[SEQ // 8, s8, pl.ds(16 * j, 16)] = zeros16
        iota = lax.iota(jnp.int32, 16)
        e_idx = [iota + 16 * k for k in range(EMB_DIM // 16)]

        def start_gather(g, sg):
            pltpu.async_copy(table_hbm.at[idx_v.at[g // 8, g % 8]],
                             gbufs[sg], gsem.at[sg])

        def wait_gather(sg):
            pltpu.make_async_copy(
                table_hbm.at[pl.ds(0, BB)], gbufs[sg], gsem.at[sg]).wait()

        def transpose(sg, st):
            src, dst = gbufs[sg], tbufs[st]

            @plsc.parallel_loop(0, BB, step=1, unroll=4)
            def body(b):
                b_idx = jnp.full((16,), b, jnp.int32)
                for k in range(EMB_DIM // 16):
                    v = src[b, pl.ds(16 * k, 16)]
                    plsc.store_scatter(dst, [e_idx[k], b_idx], v)

        def start_outs(g, st):
            for te in range(NTILES):
                pltpu.async_copy(
                    tbufs[st].at[pl.ds(te * 8, 8), pl.ds(0, BB)],
                    out_hbm.at[g, te, wid], osem.at[st])

        def wait_outs(st):
            for te in range(NTILES):
                pltpu.make_async_copy(
                    tbufs[st].at[pl.ds(te * 8, 8), pl.ds(0, BB)],
                    out_hbm.at[0, 0, 0], osem.at[st]).wait()

        for g in range(NG):
            start_gather(g, g % NG)
        for g in range(NG):
            wait_gather(g % NG)
            if g >= NT:
                wait_outs(g % NT)
            transpose(g % NG, g % NT)
            start_outs(g, g % NT)
            start_gather(g + NG, g % NG)

        def blk_body(blk, carry):
            for b in range(NG):
                g = NG + blk * NG + b
                wait_gather(b)
                wait_outs(b % NT)
                transpose(b, b % NT)
                start_outs(g, b % NT)
                start_gather(g + NG, b)
            return carry

        lax.fori_loop(0, (NSTEPS - NG) // NG, blk_body, 0)

        for sg in range(NG):
            wait_gather(sg)
        for st in range(NT):
            wait_outs(st)

    return gather_kernel


_gather = _make_gather()


def kernel(input_ids, weight):
    ids_t = (input_ids.astype(jnp.int32).T
             .reshape(SEQ // 8, 8, NW, BB).transpose(0, 2, 1, 3))
    out = _gather(weight, ids_t)
    return out.transpose(2, 4, 0, 1, 3).reshape(BATCH, SEQ, EMB_DIM)


# speedup vs baseline: 1.5846x; 1.5846x over previous
import functools

import jax
import jax.numpy as jnp
from jax import lax
from jax.experimental import pallas as pl
from jax.experimental.pallas import tpu as pltpu
from jax.experimental.pallas import tpu_sc as plsc

BATCH = 4096
SEQ = 200
EMB_DIM = 64

NC = 2
NS = 16
NW = NC * NS

BB = BATCH // NW
NG = 4
NT = 4
NSTEPS = SEQ
NTILES = EMB_DIM // 8
TB_PAD = BB + 8


def _make_gather():
    mesh = plsc.VectorSubcoreMesh(core_axis_name="c", subcore_axis_name="s")

    @functools.partial(
        pl.kernel,
        mesh=mesh,
        out_type=jax.ShapeDtypeStruct((SEQ, NTILES, NW, 8, BB), jnp.float32),
        scratch_types=(
            [pltpu.VMEM((SEQ // 8 + 1, 8, BB), jnp.int32)]
            + [pltpu.VMEM((BB, EMB_DIM), jnp.float32)] * NG
            + [pltpu.VMEM((EMB_DIM, TB_PAD), jnp.float32)] * NT
            + [pltpu.SemaphoreType.DMA((NG,)), pltpu.SemaphoreType.DMA((NT,))]
        ),
        compiler_params=pltpu.CompilerParams(
            use_tc_tiling_on_sc=False, needs_layout_passes=False),
    )
    def gather_kernel(table_hbm, ids_hbm, out_hbm, idx_v, *bufs):
        gbufs = list(bufs[:NG])
        tbufs = list(bufs[NG:NG + NT])
        gsem, osem = bufs[NG + NT], bufs[NG + NT + 1]
        wid = lax.axis_index("s") * NC + lax.axis_index("c")
        pltpu.sync_copy(ids_hbm.at[:, wid], idx_v.at[pl.ds(0, SEQ // 8)])
        zeros16 = jnp.zeros((16,), jnp.int32)
        for s8 in range(8):
            for j in range(BB // 16):
                idx_v[SEQ // 8, s8, pl.ds(16 * j, 16)] = zeros16
        iota = lax.iota(jnp.int32, 16)
        e_idx = [iota + 16 * k for k in range(EMB_DIM // 16)]

        def start_gather(g, sg):
            pltpu.async_copy(table_hbm.at[idx_v.at[g // 8, g % 8]],
                             gbufs[sg], gsem.at[sg])

        def wait_gather(sg):
            pltpu.make_async_copy(
                table_hbm.at[pl.ds(0, BB)], gbufs[sg], gsem.at[sg]).wait()

        def transpose(sg, st):
            src, dst = gbufs[sg], tbufs[st]

            @plsc.parallel_loop(0, BB, step=1, unroll=4)
            def body(b):
                b_idx = jnp.full((16,), b, jnp.int32)
                for k in range(EMB_DIM // 16):
                    v = src[b, pl.ds(16 * k, 16)]
                    plsc.store_scatter(dst, [e_idx[k], b_idx], v)

        def start_outs(g, st):
            for te in range(NTILES):
                pltpu.async_copy(
                    tbufs[st].at[pl.ds(te * 8, 8), pl.ds(0, BB)],
                    out_hbm.at[g, te, wid], osem.at[st])

        def wait_outs(st):
            for te in range(NTILES):
                pltpu.make_async_copy(
                    tbufs[st].at[pl.ds(te * 8, 8), pl.ds(0, BB)],
                    out_hbm.at[0, 0, 0], osem.at[st]).wait()

        for g in range(NG):
            start_gather(g, g % NG)
        for g in range(NG):
            wait_gather(g % NG)
            if g >= NT:
                wait_outs(g % NT)
            transpose(g % NG, g % NT)
            start_outs(g, g % NT)
            start_gather(g + NG, g % NG)

        def blk_body(blk, carry):
            for b in range(NG):
                g = NG + blk * NG + b
                wait_gather(b)
                wait_outs(b % NT)
                transpose(b, b % NT)
                start_outs(g, b % NT)
                start_gather(g + NG, b)
            return carry

        lax.fori_loop(0, (NSTEPS - NG) // NG, blk_body, 0)

        for sg in range(NG):
            wait_gather(sg)
        for st in range(NT):
            wait_outs(st)

    return gather_kernel


_gather = _make_gather()


def kernel(input_ids, weight):
    ids_t = (input_ids.astype(jnp.int32).T
             .reshape(SEQ // 8, 8, NW, BB).transpose(0, 2, 1, 3))
    out = _gather(weight, ids_t)
    return out.transpose(2, 4, 0, 1, 3).reshape(BATCH, SEQ, EMB_DIM)


# speedup vs baseline: 1.8342x; 1.1575x over previous
import functools

import jax
import jax.numpy as jnp
from jax import lax
from jax.experimental import pallas as pl
from jax.experimental.pallas import tpu as pltpu
from jax.experimental.pallas import tpu_sc as plsc

BATCH = 4096
SEQ = 200
EMB_DIM = 64

NC = 2
NS = 16
NW = NC * NS

BB = BATCH // NW
NG = 4
PF = 3
NT = 4
NSTEPS = SEQ
NTILES = EMB_DIM // 8
TB_PAD = BB + 8


def _make_gather():
    mesh = plsc.VectorSubcoreMesh(core_axis_name="c", subcore_axis_name="s")

    @functools.partial(
        pl.kernel,
        mesh=mesh,
        out_type=jax.ShapeDtypeStruct((SEQ, NTILES, NW, 8, BB), jnp.float32),
        scratch_types=(
            [pltpu.VMEM((SEQ // 8 + 1, 8, BB), jnp.int32)]
            + [pltpu.VMEM((BB, EMB_DIM), jnp.float32)] * NG
            + [pltpu.VMEM((EMB_DIM, TB_PAD), jnp.float32)] * NT
            + [pltpu.SemaphoreType.DMA((NG,)), pltpu.SemaphoreType.DMA((NT,))]
        ),
        compiler_params=pltpu.CompilerParams(
            use_tc_tiling_on_sc=False, needs_layout_passes=False),
    )
    def gather_kernel(table_hbm, ids_hbm, out_hbm, idx_v, *bufs):
        gbufs = list(bufs[:NG])
        tbufs = list(bufs[NG:NG + NT])
        gsem, osem = bufs[NG + NT], bufs[NG + NT + 1]
        wid = lax.axis_index("s") * NC + lax.axis_index("c")
        pltpu.sync_copy(ids_hbm.at[:, wid], idx_v.at[pl.ds(0, SEQ // 8)])
        zeros16 = jnp.zeros((16,), jnp.int32)
        for s8 in range(8):
            for j in range(BB // 16):
                idx_v[SEQ // 8, s8, pl.ds(16 * j, 16)] = zeros16
        iota = lax.iota(jnp.int32, 16)
        e_idx = [iota + 16 * k for k in range(EMB_DIM // 16)]

        def start_gather(g, sg):
            pltpu.async_copy(table_hbm.at[idx_v.at[g // 8, g % 8]],
                             gbufs[sg], gsem.at[sg])

        def wait_gather(sg):
            pltpu.make_async_copy(
                table_hbm.at[pl.ds(0, BB)], gbufs[sg], gsem.at[sg]).wait()

        def transpose(sg, st):
            src, dst = gbufs[sg], tbufs[st]

            @plsc.parallel_loop(0, BB, step=1, unroll=4)
            def body(b):
                b_idx = jnp.full((16,), b, jnp.int32)
                for k in range(EMB_DIM // 16):
                    v = src[b, pl.ds(16 * k, 16)]
                    plsc.store_scatter(dst, [e_idx[k], b_idx], v)

        def start_outs(g, st):
            for te in range(NTILES):
                pltpu.async_copy(
                    tbufs[st].at[pl.ds(te * 8, 8), pl.ds(0, BB)],
                    out_hbm.at[g, te, wid], osem.at[st])

        def wait_outs(st):
            for te in range(NTILES):
                pltpu.make_async_copy(
                    tbufs[st].at[pl.ds(te * 8, 8), pl.ds(0, BB)],
                    out_hbm.at[0, 0, 0], osem.at[st]).wait()

        for g in range(PF):
            start_gather(g, g % NG)
        for g in range(NG):
            wait_gather(g % NG)
            transpose(g % NG, g % NT)
            start_outs(g, g % NT)
            start_gather(g + PF, (g + PF) % NG)

        def blk_body(blk, carry):
            for b in range(NG):
                g = NG + blk * NG + b
                wait_gather(b)
                wait_outs(b % NT)
                transpose(b, b % NT)
                start_outs(g, b % NT)
                start_gather(g + PF, (b + PF) % NG)
            return carry

        lax.fori_loop(0, (NSTEPS - NG) // NG, blk_body, 0)

        for sg in range(PF):
            wait_gather((NSTEPS + sg) % NG)
        for st in range(NT):
            wait_outs(st)

    return gather_kernel


_gather = _make_gather()


def kernel(input_ids, weight):
    ids_t = (input_ids.astype(jnp.int32).T
             .reshape(SEQ // 8, 8, NW, BB).transpose(0, 2, 1, 3))
    out = _gather(weight, ids_t)
    return out.transpose(2, 4, 0, 1, 3).reshape(BATCH, SEQ, EMB_DIM)
